# Initial kernel scaffold; baseline (speedup 1.0000x reference)
import jax
import jax.numpy as jnp
from jax.experimental import pallas as pl


def kernel(node_memory, node_features, edge_features, time_encoding, edge_index_causal, edge_index_conseq, W_src, b_src, W_msg, b_msg, W_dst, b_dst, W_upd, b_upd):
    raise NotImplementedError("write your pallas kernel here")


# speedup vs baseline: 2.9753x; 2.9753x over previous
import functools

import jax
import jax.numpy as jnp
from jax import lax
from jax.experimental import pallas as pl
from jax.experimental.pallas import tpu as pltpu
from jax.experimental.pallas import tpu_sc as plsc

N = 10000
E = 320000
DM = 128
DE = 16
DT = 16
H = 64
NS = 16
NPT = 640
N2 = NS * NPT
K = 128
NB = 160
EPT = NB * K
E2 = NS * EPT

_mesh = plsc.VectorSubcoreMesh(core_axis_name="c", subcore_axis_name="s")
_sc_params = pltpu.CompilerParams(use_tc_tiling_on_sc=False)


@functools.partial(
    pl.kernel,
    out_type=jax.ShapeDtypeStruct((2, N2, H), jnp.float32),
    mesh=_mesh,
    compiler_params=_sc_params,
    scratch_types=[
        pltpu.VMEM_SHARED((N2, H), jnp.float32),
        pltpu.VMEM_SHARED((N2, H), jnp.float32),
        pltpu.VMEM((K, H), jnp.float32),
        pltpu.VMEM((K, H), jnp.float32),
        pltpu.VMEM((K,), jnp.int32),
        pltpu.VMEM((K,), jnp.int32),
        pltpu.SemaphoreType.DMA,
        pltpu.SemaphoreType.DMA,
    ],
)
def _sc_edge_pass(b_hbm, c_hbm, src_hbm, dst_hbm, z_hbm, out_hbm,
                  b_sh, agg_sh, cbuf, gbuf, sidx, didx, sem0, sem1):
    c = lax.axis_index("c")
    s = lax.axis_index("s")
    row0 = s * NPT

    pltpu.sync_copy(b_hbm.at[c, pl.ds(row0, NPT)], b_sh.at[pl.ds(row0, NPT)])
    pltpu.sync_copy(z_hbm, agg_sh.at[pl.ds(row0, NPT)])
    plsc.subcore_barrier()

    def _blk(i, carry):
        e0 = s * EPT + i * K
        pltpu.sync_copy(src_hbm.at[pl.ds(e0, K)], sidx)
        pltpu.sync_copy(dst_hbm.at[pl.ds(e0, K)], didx)
        cp_c = pltpu.async_copy(c_hbm.at[c, pl.ds(e0, K)], cbuf, sem0)
        cp_g = pltpu.async_copy(b_sh.at[sidx], gbuf, sem1)
        cp_c.wait()
        cp_g.wait()

        def _mrow(r, cr):
            for j in range(H // 16):
                sl = pl.ds(j * 16, 16)
                gbuf[r, sl] = jnp.maximum(gbuf[r, sl] + cbuf[r, sl], 0.0)
            return cr

        lax.fori_loop(0, K, _mrow, 0)
        pltpu.sync_copy(gbuf, agg_sh.at[didx], add=True)
        return carry

    lax.fori_loop(0, NB, _blk, 0)

    plsc.subcore_barrier()
    pltpu.sync_copy(agg_sh.at[pl.ds(row0, NPT)],
                    out_hbm.at[c, pl.ds(row0, NPT)])


@functools.partial(
    pl.kernel,
    out_type=jax.ShapeDtypeStruct((2, N2, 16), jnp.float32),
    mesh=_mesh,
    compiler_params=_sc_params,
    scratch_types=[
        pltpu.VMEM_SHARED((N2, 16), jnp.float32),
        pltpu.VMEM((K, 16), jnp.float32),
        pltpu.VMEM((K,), jnp.int32),
    ],
)
def _sc_counts(dst2_hbm, z_hbm, ones_hbm, out_hbm, cnt_sh, ones, idxb):
    c = lax.axis_index("c")
    s = lax.axis_index("s")
    row0 = s * NPT

    pltpu.sync_copy(z_hbm, cnt_sh.at[pl.ds(row0, NPT)])
    pltpu.sync_copy(ones_hbm, ones)
    plsc.subcore_barrier()

    def _blk(i, carry):
        e0 = s * EPT + i * K
        pltpu.sync_copy(dst2_hbm.at[c, pl.ds(e0, K)], idxb)
        pltpu.sync_copy(ones, cnt_sh.at[idxb], add=True)
        return carry

    lax.fori_loop(0, NB, _blk, 0)

    plsc.subcore_barrier()
    pltpu.sync_copy(cnt_sh.at[pl.ds(row0, NPT)],
                    out_hbm.at[c, pl.ds(row0, NPT)])


def _dot(a, b):
    return jnp.dot(a, b, preferred_element_type=jnp.float32)


def _fold_body(ws_ref, wm_ref, wd_ref, wu_ref, wc_ref, wdu_ref):
    wc_ref[...] = _dot(ws_ref[:DM, :], wm_ref[:DM, :])
    wdu_ref[...] = _dot(wd_ref[:DM, :], wu_ref[DM:, :])


def _fold_weights(w_src, w_msg, w_dst, w_upd):
    return pl.pallas_call(
        _fold_body,
        out_shape=(jax.ShapeDtypeStruct((DM, DM), jnp.float32),
                   jax.ShapeDtypeStruct((DM, DM), jnp.float32)),
    )(w_src, w_msg, w_dst, w_upd)


_RN = 1024


def _node_setup_body(mem_ref, nf_ref, ws_ref, bs_ref, wm_ref, wd_ref,
                     bd_ref, wu_ref, bu_ref, wc_ref,
                     d_ref, f_ref, b_ref):
    nf = nf_ref[...]
    d = _dot(_dot(nf, ws_ref[DM:, :]) + bs_ref[...], wm_ref[:DM, :])
    d_ref[...] = d
    f_ref[...] = _dot(_dot(nf, wd_ref[DM:, :]) + bd_ref[...],
                      wu_ref[DM:, :]) + bu_ref[...]
    b0 = _dot(mem_ref[...], wc_ref[...]) + d
    b_ref[0] = b0[:, :H]
    b_ref[1] = b0[:, H:]


def _node_setup(mem_p, nf_p, w_src, b_src2, w_msg, w_dst, b_dst2, w_upd,
                b_upd2, wc):
    grid = (N2 // _RN,)
    row_spec = pl.BlockSpec((_RN, DM), lambda i: (i, 0))
    w_specs = [
        pl.BlockSpec((DM + DM, DM), lambda i: (0, 0)),
        pl.BlockSpec((1, DM), lambda i: (0, 0)),
        pl.BlockSpec((DM + DE + DT, DM), lambda i: (0, 0)),
        pl.BlockSpec((DM + DM, DM), lambda i: (0, 0)),
        pl.BlockSpec((1, DM), lambda i: (0, 0)),
        pl.BlockSpec((DM + DM, DM), lambda i: (0, 0)),
        pl.BlockSpec((1, DM), lambda i: (0, 0)),
        pl.BlockSpec((DM, DM), lambda i: (0, 0)),
    ]
    return pl.pallas_call(
        _node_setup_body,
        grid=grid,
        in_specs=[row_spec, row_spec] + w_specs,
        out_specs=(row_spec, row_spec,
                   pl.BlockSpec((2, _RN, H), lambda i: (0, i, 0))),
        out_shape=(jax.ShapeDtypeStruct((N2, DM), jnp.float32),
                   jax.ShapeDtypeStruct((N2, DM), jnp.float32),
                   jax.ShapeDtypeStruct((2, N2, H), jnp.float32)),
    )(mem_p, nf_p, w_src, b_src2, w_msg, w_dst, b_dst2, w_upd, b_upd2, wc)


_RE = 2048


def _edge_setup_body(ef_ref, te_ref, wm_ref, bm_ref, c_ref):
    cfull = (_dot(ef_ref[...], wm_ref[DM:DM + DE, :]) +
             _dot(te_ref[...], wm_ref[DM + DE:, :]) + bm_ref[...])
    c_ref[0] = cfull[:, :H]
    c_ref[1] = cfull[:, H:]


def _edge_setup(ef, te, w_msg, b_msg2):
    grid = (E2 // _RE,)
    return pl.pallas_call(
        _edge_setup_body,
        grid=grid,
        in_specs=[
            pl.BlockSpec((_RE, DE), lambda i: (i, 0)),
            pl.BlockSpec((_RE, DT), lambda i: (i, 0)),
            pl.BlockSpec((DM + DE + DT, DM), lambda i: (0, 0)),
            pl.BlockSpec((1, DM), lambda i: (0, 0)),
        ],
        out_specs=pl.BlockSpec((2, _RE, H), lambda i: (0, i, 0)),
        out_shape=jax.ShapeDtypeStruct((2, E2, H), jnp.float32),
    )(ef, te, w_msg, b_msg2)


def _layer_body(agg_ref, cnt_ref, mem_ref, d_ref, f_ref, wu_ref, wdu_ref,
                wc_ref, mem_out_ref, b_out_ref):
    inv = 1.0 / jnp.maximum(cnt_ref[:, 0:1], 1.0)
    a0 = agg_ref[0] * inv
    a1 = agg_ref[1] * inv
    pre = (_dot(a0, wu_ref[:H, :]) + _dot(a1, wu_ref[H:DM, :]) +
           _dot(mem_ref[...], wdu_ref[...]) + f_ref[...])
    m2 = jnp.tanh(pre)
    mem_out_ref[...] = m2
    bn = _dot(m2, wc_ref[...]) + d_ref[...]
    b_out_ref[0] = bn[:, :H]
    b_out_ref[1] = bn[:, H:]


def _layer_update(agg, cnt_l, mem_p, d, f, w_upd, wdu, wc):
    grid = (N2 // _RN,)
    row_spec = pl.BlockSpec((_RN, DM), lambda i: (i, 0))
    return pl.pallas_call(
        _layer_body,
        grid=grid,
        in_specs=[
            pl.BlockSpec((2, _RN, H), lambda i: (0, i, 0)),
            pl.BlockSpec((_RN, 16), lambda i: (i, 0)),
            row_spec,
            row_spec,
            row_spec,
            pl.BlockSpec((DM + DM, DM), lambda i: (0, 0)),
            pl.BlockSpec((DM, DM), lambda i: (0, 0)),
            pl.BlockSpec((DM, DM), lambda i: (0, 0)),
        ],
        out_specs=(row_spec, pl.BlockSpec((2, _RN, H), lambda i: (0, i, 0))),
        out_shape=(jax.ShapeDtypeStruct((N2, DM), jnp.float32),
                   jax.ShapeDtypeStruct((2, N2, H), jnp.float32)),
    )(agg, cnt_l, mem_p, d, f, w_upd, wdu, wc)


def kernel(node_memory, node_features, edge_features, time_encoding,
           edge_index_causal, edge_index_conseq,
           W_src, b_src, W_msg, b_msg, W_dst, b_dst, W_upd, b_upd):
    pad = ((0, N2 - N), (0, 0))
    mem_p = jnp.pad(node_memory, pad)
    nf_p = jnp.pad(node_features, pad)
    b_src2 = b_src.reshape(1, DM)
    b_msg2 = b_msg.reshape(1, DM)
    b_dst2 = b_dst.reshape(1, DM)
    b_upd2 = b_upd.reshape(1, DM)

    epad = E2 - E
    src_c = jnp.concatenate([edge_index_causal[0],
                             jnp.zeros((epad,), jnp.int32)])
    dst_c = jnp.concatenate([edge_index_causal[1],
                             jnp.full((epad,), N, jnp.int32)])
    src_q = jnp.concatenate([edge_index_conseq[0],
                             jnp.zeros((epad,), jnp.int32)])
    dst_q = jnp.concatenate([edge_index_conseq[1],
                             jnp.full((epad,), N, jnp.int32)])
    ef_p = jnp.pad(edge_features, ((0, epad), (0, 0)))
    te_p = jnp.pad(time_encoding, ((0, epad), (0, 0)))

    zeros_nh = jnp.zeros((NPT, H), jnp.float32)
    zeros_n16 = jnp.zeros((NPT, 16), jnp.float32)
    ones_k16 = jnp.ones((K, 16), jnp.float32)

    wc, wdu = _fold_weights(W_src, W_msg, W_dst, W_upd)
    d, f, b = _node_setup(mem_p, nf_p, W_src, b_src2, W_msg, W_dst, b_dst2,
                          W_upd, b_upd2, wc)
    c_edge = _edge_setup(ef_p, te_p, W_msg, b_msg2)
    dst2 = jnp.stack([dst_c, dst_q])
    cnt = _sc_counts(dst2, zeros_n16, ones_k16)

    for layer in range(4):
        if layer % 2 == 0:
            src_l, dst_l = src_c, dst_c
        else:
            src_l, dst_l = src_q, dst_q
        agg = _sc_edge_pass(b, c_edge, src_l, dst_l, zeros_nh)
        mem_p, b = _layer_update(agg, cnt[layer % 2], mem_p, d, f, W_upd,
                                 wdu, wc)
    return mem_p[:N]


# speedup vs baseline: 4.0928x; 1.3756x over previous
import functools

import jax
import jax.numpy as jnp
from jax import lax
from jax.experimental import pallas as pl
from jax.experimental.pallas import tpu as pltpu
from jax.experimental.pallas import tpu_sc as plsc

N = 10000
E = 320000
DM = 128
DE = 16
DT = 16
H = 64
NS = 16
NPT = 640
N2 = NS * NPT
K = 128
NB = 160
EPT = NB * K
E2 = NS * EPT

_mesh = plsc.VectorSubcoreMesh(core_axis_name="c", subcore_axis_name="s")
_sc_params = pltpu.CompilerParams(use_tc_tiling_on_sc=False)


@functools.partial(
    pl.kernel,
    out_type=jax.ShapeDtypeStruct((2, N2, H), jnp.float32),
    mesh=_mesh,
    compiler_params=_sc_params,
    scratch_types=[
        pltpu.VMEM_SHARED((N2, H), jnp.float32),
        pltpu.VMEM_SHARED((N2, H), jnp.float32),
        pltpu.VMEM((K, H), jnp.float32),
        pltpu.VMEM((K, H), jnp.float32),
        pltpu.VMEM((K, H), jnp.float32),
        pltpu.VMEM((K, H), jnp.float32),
        pltpu.VMEM((K,), jnp.int32),
        pltpu.VMEM((K,), jnp.int32),
        pltpu.VMEM((K,), jnp.int32),
        pltpu.VMEM((K,), jnp.int32),
        pltpu.SemaphoreType.DMA,
        pltpu.SemaphoreType.DMA,
        pltpu.SemaphoreType.DMA,
        pltpu.SemaphoreType.DMA,
    ],
)
def _sc_edge_pass(b_hbm, c_hbm, src_hbm, dst_hbm, z_hbm, out_hbm,
                  b_sh, agg_sh, cbuf0, cbuf1, gbuf0, gbuf1,
                  sidx0, sidx1, didx0, didx1, semf0, semf1, semg0, semg1):
    c = lax.axis_index("c")
    s = lax.axis_index("s")
    row0 = s * NPT
    slots = ((cbuf0, gbuf0, sidx0, didx0, semf0, semg0),
             (cbuf1, gbuf1, sidx1, didx1, semf1, semg1))
    NBH = NB // 2

    def _fetch(i, slot):
        cbuf, _, sidx, didx, semf, _ = slot
        e0 = s * EPT + i * K
        return (pltpu.make_async_copy(src_hbm.at[pl.ds(e0, K)], sidx, semf),
                pltpu.make_async_copy(dst_hbm.at[pl.ds(e0, K)], didx, semf),
                pltpu.make_async_copy(c_hbm.at[c, pl.ds(e0, K)], cbuf, semf))

    pltpu.sync_copy(b_hbm.at[c, pl.ds(row0, NPT)], b_sh.at[pl.ds(row0, NPT)])
    pltpu.sync_copy(z_hbm, agg_sh.at[pl.ds(row0, NPT)])
    plsc.subcore_barrier()

    for cp in _fetch(0, slots[0]) + _fetch(1, slots[1]):
        cp.start()

    def _compute(slot):
        cbuf, gbuf, _, didx, _, _ = slot

        def _mrow(r, cr):
            for j in range(H // 16):
                sl = pl.ds(j * 16, 16)
                gbuf[r, sl] = jnp.maximum(gbuf[r, sl] + cbuf[r, sl], 0.0)
            return cr

        lax.fori_loop(0, K, _mrow, 0)
        pltpu.sync_copy(gbuf, agg_sh.at[didx], add=True)

    def _blk(j, carry):
        gathers = []
        for t, slot in enumerate(slots):
            for cp in _fetch(2 * j + t, slot):
                cp.wait()
            gathers.append(
                pltpu.async_copy(b_sh.at[slot[2]], slot[1], slot[5]))
        for t, slot in enumerate(slots):
            gathers[t].wait()
            _compute(slot)

            @pl.when(j < NBH - 1)
            def _():
                for cp in _fetch(2 * j + 2 + t, slot):
                    cp.start()

        return carry

    lax.fori_loop(0, NBH, _blk, 0)

    plsc.subcore_barrier()
    pltpu.sync_copy(agg_sh.at[pl.ds(row0, NPT)],
                    out_hbm.at[c, pl.ds(row0, NPT)])


@functools.partial(
    pl.kernel,
    out_type=jax.ShapeDtypeStruct((2, N2, 16), jnp.float32),
    mesh=_mesh,
    compiler_params=_sc_params,
    scratch_types=[
        pltpu.VMEM_SHARED((N2, 16), jnp.float32),
        pltpu.VMEM((K, 16), jnp.float32),
        pltpu.VMEM((K,), jnp.int32),
        pltpu.VMEM((K,), jnp.int32),
        pltpu.SemaphoreType.DMA,
        pltpu.SemaphoreType.DMA,
    ],
)
def _sc_counts(dst2_hbm, z_hbm, ones_hbm, out_hbm, cnt_sh, ones,
               idx0, idx1, semi0, semi1):
    c = lax.axis_index("c")
    s = lax.axis_index("s")
    row0 = s * NPT
    slots = ((idx0, semi0), (idx1, semi1))
    NBH = NB // 2

    def _fetch(i, slot):
        e0 = s * EPT + i * K
        return pltpu.make_async_copy(dst2_hbm.at[c, pl.ds(e0, K)],
                                     slot[0], slot[1])

    pltpu.sync_copy(z_hbm, cnt_sh.at[pl.ds(row0, NPT)])
    pltpu.sync_copy(ones_hbm, ones)
    plsc.subcore_barrier()

    _fetch(0, slots[0]).start()
    _fetch(1, slots[1]).start()

    def _blk(j, carry):
        for t, slot in enumerate(slots):
            _fetch(2 * j + t, slot).wait()
            pltpu.sync_copy(ones, cnt_sh.at[slot[0]], add=True)

            @pl.when(j < NBH - 1)
            def _():
                _fetch(2 * j + 2 + t, slot).start()

        return carry

    lax.fori_loop(0, NBH, _blk, 0)

    plsc.subcore_barrier()
    pltpu.sync_copy(cnt_sh.at[pl.ds(row0, NPT)],
                    out_hbm.at[c, pl.ds(row0, NPT)])


def _dot(a, b):
    return jnp.dot(a, b, preferred_element_type=jnp.float32)


def _fold_body(ws_ref, wm_ref, wd_ref, wu_ref, wc_ref, wdu_ref):
    wc_ref[...] = _dot(ws_ref[:DM, :], wm_ref[:DM, :])
    wdu_ref[...] = _dot(wd_ref[:DM, :], wu_ref[DM:, :])


def _fold_weights(w_src, w_msg, w_dst, w_upd):
    return pl.pallas_call(
        _fold_body,
        out_shape=(jax.ShapeDtypeStruct((DM, DM), jnp.float32),
                   jax.ShapeDtypeStruct((DM, DM), jnp.float32)),
    )(w_src, w_msg, w_dst, w_upd)


_RN = 1024


def _node_setup_body(mem_ref, nf_ref, ws_ref, bs_ref, wm_ref, wd_ref,
                     bd_ref, wu_ref, bu_ref, wc_ref,
                     d_ref, f_ref, b_ref):
    nf = nf_ref[...]
    d = _dot(_dot(nf, ws_ref[DM:, :]) + bs_ref[...], wm_ref[:DM, :])
    d_ref[...] = d
    f_ref[...] = _dot(_dot(nf, wd_ref[DM:, :]) + bd_ref[...],
                      wu_ref[DM:, :]) + bu_ref[...]
    b0 = _dot(mem_ref[...], wc_ref[...]) + d
    b_ref[0] = b0[:, :H]
    b_ref[1] = b0[:, H:]


def _node_setup(mem_p, nf_p, w_src, b_src2, w_msg, w_dst, b_dst2, w_upd,
                b_upd2, wc):
    grid = (N2 // _RN,)
    row_spec = pl.BlockSpec((_RN, DM), lambda i: (i, 0))
    w_specs = [
        pl.BlockSpec((DM + DM, DM), lambda i: (0, 0)),
        pl.BlockSpec((1, DM), lambda i: (0, 0)),
        pl.BlockSpec((DM + DE + DT, DM), lambda i: (0, 0)),
        pl.BlockSpec((DM + DM, DM), lambda i: (0, 0)),
        pl.BlockSpec((1, DM), lambda i: (0, 0)),
        pl.BlockSpec((DM + DM, DM), lambda i: (0, 0)),
        pl.BlockSpec((1, DM), lambda i: (0, 0)),
        pl.BlockSpec((DM, DM), lambda i: (0, 0)),
    ]
    return pl.pallas_call(
        _node_setup_body,
        grid=grid,
        in_specs=[row_spec, row_spec] + w_specs,
        out_specs=(row_spec, row_spec,
                   pl.BlockSpec((2, _RN, H), lambda i: (0, i, 0))),
        out_shape=(jax.ShapeDtypeStruct((N2, DM), jnp.float32),
                   jax.ShapeDtypeStruct((N2, DM), jnp.float32),
                   jax.ShapeDtypeStruct((2, N2, H), jnp.float32)),
    )(mem_p, nf_p, w_src, b_src2, w_msg, w_dst, b_dst2, w_upd, b_upd2, wc)


_RE = 2048


def _edge_setup_body(ef_ref, te_ref, wm_ref, bm_ref, c_ref):
    cfull = (_dot(ef_ref[...], wm_ref[DM:DM + DE, :]) +
             _dot(te_ref[...], wm_ref[DM + DE:, :]) + bm_ref[...])
    c_ref[0] = cfull[:, :H]
    c_ref[1] = cfull[:, H:]


def _edge_setup(ef, te, w_msg, b_msg2):
    grid = (E2 // _RE,)
    return pl.pallas_call(
        _edge_setup_body,
        grid=grid,
        in_specs=[
            pl.BlockSpec((_RE, DE), lambda i: (i, 0)),
            pl.BlockSpec((_RE, DT), lambda i: (i, 0)),
            pl.BlockSpec((DM + DE + DT, DM), lambda i: (0, 0)),
            pl.BlockSpec((1, DM), lambda i: (0, 0)),
        ],
        out_specs=pl.BlockSpec((2, _RE, H), lambda i: (0, i, 0)),
        out_shape=jax.ShapeDtypeStruct((2, E2, H), jnp.float32),
    )(ef, te, w_msg, b_msg2)


def _layer_body(agg_ref, cnt_ref, mem_ref, d_ref, f_ref, wu_ref, wdu_ref,
                wc_ref, mem_out_ref, b_out_ref):
    inv = 1.0 / jnp.maximum(cnt_ref[:, 0:1], 1.0)
    a0 = agg_ref[0] * inv
    a1 = agg_ref[1] * inv
    pre = (_dot(a0, wu_ref[:H, :]) + _dot(a1, wu_ref[H:DM, :]) +
           _dot(mem_ref[...], wdu_ref[...]) + f_ref[...])
    m2 = jnp.tanh(pre)
    mem_out_ref[...] = m2
    bn = _dot(m2, wc_ref[...]) + d_ref[...]
    b_out_ref[0] = bn[:, :H]
    b_out_ref[1] = bn[:, H:]


def _layer_update(agg, cnt_l, mem_p, d, f, w_upd, wdu, wc):
    grid = (N2 // _RN,)
    row_spec = pl.BlockSpec((_RN, DM), lambda i: (i, 0))
    return pl.pallas_call(
        _layer_body,
        grid=grid,
        in_specs=[
            pl.BlockSpec((2, _RN, H), lambda i: (0, i, 0)),
            pl.BlockSpec((_RN, 16), lambda i: (i, 0)),
            row_spec,
            row_spec,
            row_spec,
            pl.BlockSpec((DM + DM, DM), lambda i: (0, 0)),
            pl.BlockSpec((DM, DM), lambda i: (0, 0)),
            pl.BlockSpec((DM, DM), lambda i: (0, 0)),
        ],
        out_specs=(row_spec, pl.BlockSpec((2, _RN, H), lambda i: (0, i, 0))),
        out_shape=(jax.ShapeDtypeStruct((N2, DM), jnp.float32),
                   jax.ShapeDtypeStruct((2, N2, H), jnp.float32)),
    )(agg, cnt_l, mem_p, d, f, w_upd, wdu, wc)


def kernel(node_memory, node_features, edge_features, time_encoding,
           edge_index_causal, edge_index_conseq,
           W_src, b_src, W_msg, b_msg, W_dst, b_dst, W_upd, b_upd):
    pad = ((0, N2 - N), (0, 0))
    mem_p = jnp.pad(node_memory, pad)
    nf_p = jnp.pad(node_features, pad)
    b_src2 = b_src.reshape(1, DM)
    b_msg2 = b_msg.reshape(1, DM)
    b_dst2 = b_dst.reshape(1, DM)
    b_upd2 = b_upd.reshape(1, DM)

    epad = E2 - E
    src_c = jnp.concatenate([edge_index_causal[0],
                             jnp.zeros((epad,), jnp.int32)])
    dst_c = jnp.concatenate([edge_index_causal[1],
                             jnp.full((epad,), N, jnp.int32)])
    src_q = jnp.concatenate([edge_index_conseq[0],
                             jnp.zeros((epad,), jnp.int32)])
    dst_q = jnp.concatenate([edge_index_conseq[1],
                             jnp.full((epad,), N, jnp.int32)])
    ef_p = jnp.pad(edge_features, ((0, epad), (0, 0)))
    te_p = jnp.pad(time_encoding, ((0, epad), (0, 0)))

    zeros_nh = jnp.zeros((NPT, H), jnp.float32)
    zeros_n16 = jnp.zeros((NPT, 16), jnp.float32)
    ones_k16 = jnp.ones((K, 16), jnp.float32)

    wc, wdu = _fold_weights(W_src, W_msg, W_dst, W_upd)
    d, f, b = _node_setup(mem_p, nf_p, W_src, b_src2, W_msg, W_dst, b_dst2,
                          W_upd, b_upd2, wc)
    c_edge = _edge_setup(ef_p, te_p, W_msg, b_msg2)
    dst2 = jnp.stack([dst_c, dst_q])
    cnt = _sc_counts(dst2, zeros_n16, ones_k16)

    for layer in range(4):
        if layer % 2 == 0:
            src_l, dst_l = src_c, dst_c
        else:
            src_l, dst_l = src_q, dst_q
        agg = _sc_edge_pass(b, c_edge, src_l, dst_l, zeros_nh)
        mem_p, b = _layer_update(agg, cnt[layer % 2], mem_p, d, f, W_upd,
                                 wdu, wc)
    return mem_p[:N]


# speedup vs baseline: 4.7930x; 1.1711x over previous
import functools

import jax
import jax.numpy as jnp
from jax import lax
from jax.experimental import pallas as pl
from jax.experimental.pallas import tpu as pltpu
from jax.experimental.pallas import tpu_sc as plsc

N = 10000
E = 320000
DM = 128
DE = 16
DT = 16
H = 64
NS = 16
NPT = 640
N2 = NS * NPT
K = 128
NB = 160
EPT = NB * K
E2 = NS * EPT

_mesh = plsc.VectorSubcoreMesh(core_axis_name="c", subcore_axis_name="s")
_sc_params = pltpu.CompilerParams(use_tc_tiling_on_sc=False)


@functools.partial(
    pl.kernel,
    out_type=jax.ShapeDtypeStruct((2, N2, H), jnp.float32),
    mesh=_mesh,
    compiler_params=_sc_params,
    scratch_types=[
        pltpu.VMEM_SHARED((N2, H), jnp.float32),
        pltpu.VMEM_SHARED((N2, H), jnp.float32),
        pltpu.VMEM((K, H), jnp.float32),
        pltpu.VMEM((K, H), jnp.float32),
        pltpu.VMEM((K, H), jnp.float32),
        pltpu.VMEM((K, H), jnp.float32),
        pltpu.VMEM((K,), jnp.int32),
        pltpu.VMEM((K,), jnp.int32),
        pltpu.VMEM((K,), jnp.int32),
        pltpu.VMEM((K,), jnp.int32),
        pltpu.VMEM((K,), jnp.int32),
        pltpu.VMEM((K,), jnp.int32),
        pltpu.SemaphoreType.DMA,
        pltpu.SemaphoreType.DMA,
        pltpu.SemaphoreType.DMA,
        pltpu.SemaphoreType.DMA,
        pltpu.SemaphoreType.DMA,
        pltpu.SemaphoreType.DMA,
    ],
)
def _sc_edge_pass(b_hbm, c_hbm, src_hbm, dst_hbm, z_hbm, out_hbm,
                  b_sh, agg_sh, cbuf0, cbuf1, gbuf0, gbuf1,
                  sidx0, sidx1, didx0, didx1, didx2, didx3,
                  semf0, semf1, semg0, semg1, sems0, sems1):
    c = lax.axis_index("c")
    s = lax.axis_index("s")
    row0 = s * NPT
    slots = ((cbuf0, gbuf0, sidx0, semf0, semg0, sems0),
             (cbuf1, gbuf1, sidx1, semf1, semg1, sems1))
    dring = (didx0, didx1, didx2, didx3)
    NQ = NB // 4

    def _fetch(e0, ru):
        slot = slots[ru % 2]
        return (pltpu.make_async_copy(src_hbm.at[pl.ds(e0, K)], slot[2],
                                      slot[3]),
                pltpu.make_async_copy(dst_hbm.at[pl.ds(e0, K)], dring[ru],
                                      slot[3]),
                pltpu.make_async_copy(c_hbm.at[c, pl.ds(e0, K)], slot[0],
                                      slot[3]))

    def _scatter_start(ru):
        slot = slots[ru % 2]
        pltpu.async_copy(slot[1], agg_sh.at[dring[ru]], slot[5], add=True)

    def _scatter_wait(ru):
        slot = slots[ru % 2]
        pltpu.make_async_copy(slot[1], agg_sh.at[dring[ru]], slot[5]).wait()

    pltpu.sync_copy(b_hbm.at[c, pl.ds(row0, NPT)], b_sh.at[pl.ds(row0, NPT)])
    pltpu.sync_copy(z_hbm, agg_sh.at[pl.ds(row0, NPT)])
    plsc.subcore_barrier()

    tile0 = s * EPT
    for cp in _fetch(tile0, 0) + _fetch(tile0 + K, 1):
        cp.start()

    def _relu_add(slot):
        cbuf, gbuf = slot[0], slot[1]

        def _mrow(r4, cr):
            r = r4 * 4
            for rr in range(4):
                for jj in range(H // 16):
                    sl = pl.ds(jj * 16, 16)
                    gbuf[r + rr, sl] = jnp.maximum(
                        gbuf[r + rr, sl] + cbuf[r + rr, sl], 0.0)
            return cr

        lax.fori_loop(0, K // 4, _mrow, 0)

    def _blk(q, carry):
        base = tile0 + q * 4 * K
        for u0 in (0, 2):
            gathers = []
            for u in (u0, u0 + 1):
                slot = slots[u % 2]
                for cp in _fetch(base + u * K, u):
                    cp.wait()
                if u0 == 0:
                    @pl.when(q > 0)
                    def _():
                        _scatter_wait(u + 2)
                else:
                    _scatter_wait(u - 2)
                gathers.append(
                    pltpu.async_copy(b_sh.at[slot[2]], slot[1], slot[4]))
            for u in (u0, u0 + 1):
                slot = slots[u % 2]
                gathers[u - u0].wait()
                _relu_add(slot)
                _scatter_start(u)
                if u0 == 0:
                    for cp in _fetch(base + (u + 2) * K, u + 2):
                        cp.start()
                else:
                    @pl.when(q < NQ - 1)
                    def _():
                        for cp in _fetch(base + (u + 2) * K, (u + 2) % 4):
                            cp.start()

        return carry

    lax.fori_loop(0, NQ, _blk, 0)
    _scatter_wait(2)
    _scatter_wait(3)

    plsc.subcore_barrier()
    pltpu.sync_copy(agg_sh.at[pl.ds(row0, NPT)],
                    out_hbm.at[c, pl.ds(row0, NPT)])


@functools.partial(
    pl.kernel,
    out_type=jax.ShapeDtypeStruct((2, N2, 16), jnp.float32),
    mesh=_mesh,
    compiler_params=_sc_params,
    scratch_types=[
        pltpu.VMEM_SHARED((N2, 16), jnp.float32),
        pltpu.VMEM((K, 16), jnp.float32),
        pltpu.VMEM((K,), jnp.int32),
        pltpu.VMEM((K,), jnp.int32),
        pltpu.SemaphoreType.DMA,
        pltpu.SemaphoreType.DMA,
    ],
)
def _sc_counts(dst2_hbm, z_hbm, ones_hbm, out_hbm, cnt_sh, ones,
               idx0, idx1, semi0, semi1):
    c = lax.axis_index("c")
    s = lax.axis_index("s")
    row0 = s * NPT
    slots = ((idx0, semi0), (idx1, semi1))
    NBH = NB // 2

    def _fetch(i, slot):
        e0 = s * EPT + i * K
        return pltpu.make_async_copy(dst2_hbm.at[c, pl.ds(e0, K)],
                                     slot[0], slot[1])

    pltpu.sync_copy(z_hbm, cnt_sh.at[pl.ds(row0, NPT)])
    pltpu.sync_copy(ones_hbm, ones)
    plsc.subcore_barrier()

    _fetch(0, slots[0]).start()
    _fetch(1, slots[1]).start()

    def _blk(j, carry):
        for t, slot in enumerate(slots):
            _fetch(2 * j + t, slot).wait()
            pltpu.sync_copy(ones, cnt_sh.at[slot[0]], add=True)

            @pl.when(j < NBH - 1)
            def _():
                _fetch(2 * j + 2 + t, slot).start()

        return carry

    lax.fori_loop(0, NBH, _blk, 0)

    plsc.subcore_barrier()
    pltpu.sync_copy(cnt_sh.at[pl.ds(row0, NPT)],
                    out_hbm.at[c, pl.ds(row0, NPT)])


def _dot(a, b):
    return jnp.dot(a, b, preferred_element_type=jnp.float32)


def _fold_body(ws_ref, wm_ref, wd_ref, wu_ref, wc_ref, wdu_ref):
    wc_ref[...] = _dot(ws_ref[:DM, :], wm_ref[:DM, :])
    wdu_ref[...] = _dot(wd_ref[:DM, :], wu_ref[DM:, :])


def _fold_weights(w_src, w_msg, w_dst, w_upd):
    return pl.pallas_call(
        _fold_body,
        out_shape=(jax.ShapeDtypeStruct((DM, DM), jnp.float32),
                   jax.ShapeDtypeStruct((DM, DM), jnp.float32)),
    )(w_src, w_msg, w_dst, w_upd)


_RN = 1024


def _node_setup_body(mem_ref, nf_ref, ws_ref, bs_ref, wm_ref, wd_ref,
                     bd_ref, wu_ref, bu_ref, wc_ref,
                     d_ref, f_ref, b_ref):
    nf = nf_ref[...]
    d = _dot(_dot(nf, ws_ref[DM:, :]) + bs_ref[...], wm_ref[:DM, :])
    d_ref[...] = d
    f_ref[...] = _dot(_dot(nf, wd_ref[DM:, :]) + bd_ref[...],
                      wu_ref[DM:, :]) + bu_ref[...]
    b0 = _dot(mem_ref[...], wc_ref[...]) + d
    b_ref[0] = b0[:, :H]
    b_ref[1] = b0[:, H:]


def _node_setup(mem_p, nf_p, w_src, b_src2, w_msg, w_dst, b_dst2, w_upd,
                b_upd2, wc):
    grid = (N2 // _RN,)
    row_spec = pl.BlockSpec((_RN, DM), lambda i: (i, 0))
    w_specs = [
        pl.BlockSpec((DM + DM, DM), lambda i: (0, 0)),
        pl.BlockSpec((1, DM), lambda i: (0, 0)),
        pl.BlockSpec((DM + DE + DT, DM), lambda i: (0, 0)),
        pl.BlockSpec((DM + DM, DM), lambda i: (0, 0)),
        pl.BlockSpec((1, DM), lambda i: (0, 0)),
        pl.BlockSpec((DM + DM, DM), lambda i: (0, 0)),
        pl.BlockSpec((1, DM), lambda i: (0, 0)),
        pl.BlockSpec((DM, DM), lambda i: (0, 0)),
    ]
    return pl.pallas_call(
        _node_setup_body,
        grid=grid,
        in_specs=[row_spec, row_spec] + w_specs,
        out_specs=(row_spec, row_spec,
                   pl.BlockSpec((2, _RN, H), lambda i: (0, i, 0))),
        out_shape=(jax.ShapeDtypeStruct((N2, DM), jnp.float32),
                   jax.ShapeDtypeStruct((N2, DM), jnp.float32),
                   jax.ShapeDtypeStruct((2, N2, H), jnp.float32)),
    )(mem_p, nf_p, w_src, b_src2, w_msg, w_dst, b_dst2, w_upd, b_upd2, wc)


_RE = 1600


def _edge_setup_body(ef_ref, te_ref, wm_ref, bm_ref, c_ref):
    cfull = (_dot(ef_ref[...], wm_ref[DM:DM + DE, :]) +
             _dot(te_ref[...], wm_ref[DM + DE:, :]) + bm_ref[...])
    c_ref[0] = cfull[:, :H]
    c_ref[1] = cfull[:, H:]


def _edge_setup(ef, te, w_msg, b_msg2):
    grid = (E // _RE,)
    return pl.pallas_call(
        _edge_setup_body,
        grid=grid,
        in_specs=[
            pl.BlockSpec((_RE, DE), lambda i: (i, 0)),
            pl.BlockSpec((_RE, DT), lambda i: (i, 0)),
            pl.BlockSpec((DM + DE + DT, DM), lambda i: (0, 0)),
            pl.BlockSpec((1, DM), lambda i: (0, 0)),
        ],
        out_specs=pl.BlockSpec((2, _RE, H), lambda i: (0, i, 0)),
        out_shape=jax.ShapeDtypeStruct((2, E2, H), jnp.float32),
    )(ef, te, w_msg, b_msg2)


def _layer_body(agg_ref, cnt_ref, mem_ref, d_ref, f_ref, wu_ref, wdu_ref,
                wc_ref, mem_out_ref, b_out_ref):
    inv = 1.0 / jnp.maximum(cnt_ref[:, 0:1], 1.0)
    a0 = agg_ref[0] * inv
    a1 = agg_ref[1] * inv
    pre = (_dot(a0, wu_ref[:H, :]) + _dot(a1, wu_ref[H:DM, :]) +
           _dot(mem_ref[...], wdu_ref[...]) + f_ref[...])
    m2 = jnp.tanh(pre)
    mem_out_ref[...] = m2
    bn = _dot(m2, wc_ref[...]) + d_ref[...]
    b_out_ref[0] = bn[:, :H]
    b_out_ref[1] = bn[:, H:]


def _layer_update(agg, cnt_l, mem_p, d, f, w_upd, wdu, wc):
    grid = (N2 // _RN,)
    row_spec = pl.BlockSpec((_RN, DM), lambda i: (i, 0))
    return pl.pallas_call(
        _layer_body,
        grid=grid,
        in_specs=[
            pl.BlockSpec((2, _RN, H), lambda i: (0, i, 0)),
            pl.BlockSpec((_RN, 16), lambda i: (i, 0)),
            row_spec,
            row_spec,
            row_spec,
            pl.BlockSpec((DM + DM, DM), lambda i: (0, 0)),
            pl.BlockSpec((DM, DM), lambda i: (0, 0)),
            pl.BlockSpec((DM, DM), lambda i: (0, 0)),
        ],
        out_specs=(row_spec, pl.BlockSpec((2, _RN, H), lambda i: (0, i, 0))),
        out_shape=(jax.ShapeDtypeStruct((N2, DM), jnp.float32),
                   jax.ShapeDtypeStruct((2, N2, H), jnp.float32)),
    )(agg, cnt_l, mem_p, d, f, w_upd, wdu, wc)


def kernel(node_memory, node_features, edge_features, time_encoding,
           edge_index_causal, edge_index_conseq,
           W_src, b_src, W_msg, b_msg, W_dst, b_dst, W_upd, b_upd):
    pad = ((0, N2 - N), (0, 0))
    mem_p = jnp.pad(node_memory, pad)
    nf_p = jnp.pad(node_features, pad)
    b_src2 = b_src.reshape(1, DM)
    b_msg2 = b_msg.reshape(1, DM)
    b_dst2 = b_dst.reshape(1, DM)
    b_upd2 = b_upd.reshape(1, DM)

    epad = E2 - E
    src_c = jnp.concatenate([edge_index_causal[0],
                             jnp.zeros((epad,), jnp.int32)])
    dst_c = jnp.concatenate([edge_index_causal[1],
                             jnp.full((epad,), N, jnp.int32)])
    src_q = jnp.concatenate([edge_index_conseq[0],
                             jnp.zeros((epad,), jnp.int32)])
    dst_q = jnp.concatenate([edge_index_conseq[1],
                             jnp.full((epad,), N, jnp.int32)])
    zeros_nh = jnp.zeros((NPT, H), jnp.float32)
    zeros_n16 = jnp.zeros((NPT, 16), jnp.float32)
    ones_k16 = jnp.ones((K, 16), jnp.float32)

    wc, wdu = _fold_weights(W_src, W_msg, W_dst, W_upd)
    d, f, b = _node_setup(mem_p, nf_p, W_src, b_src2, W_msg, W_dst, b_dst2,
                          W_upd, b_upd2, wc)
    c_edge = _edge_setup(edge_features, time_encoding, W_msg, b_msg2)
    dst2 = jnp.stack([dst_c, dst_q])
    cnt = _sc_counts(dst2, zeros_n16, ones_k16)

    for layer in range(4):
        if layer % 2 == 0:
            src_l, dst_l = src_c, dst_c
        else:
            src_l, dst_l = src_q, dst_q
        agg = _sc_edge_pass(b, c_edge, src_l, dst_l, zeros_nh)
        mem_p, b = _layer_update(agg, cnt[layer % 2], mem_p, d, f, W_upd,
                                 wdu, wc)
    return mem_p[:N]


# speedup vs baseline: 5.7345x; 1.1964x over previous
import functools

import jax
import jax.numpy as jnp
from jax import lax
from jax.experimental import pallas as pl
from jax.experimental.pallas import tpu as pltpu
from jax.experimental.pallas import tpu_sc as plsc

N = 10000
E = 320000
DM = 128
DE = 16
DT = 16
H = 64
NS = 16
NPT = 640
N2 = NS * NPT
K = 128
NB = 160
EPT = NB * K
E2 = NS * EPT

_mesh = plsc.VectorSubcoreMesh(core_axis_name="c", subcore_axis_name="s")
_sc_params = pltpu.CompilerParams(use_tc_tiling_on_sc=False)


@functools.partial(
    pl.kernel,
    out_type=jax.ShapeDtypeStruct((2, N2, H), jnp.float32),
    mesh=_mesh,
    compiler_params=_sc_params,
    scratch_types=[
        pltpu.VMEM_SHARED((N2, H), jnp.float32),
        pltpu.VMEM_SHARED((N2, H), jnp.float32),
        pltpu.VMEM((K, H), jnp.float32),
        pltpu.VMEM((K, H), jnp.float32),
        pltpu.VMEM((K, H), jnp.float32),
        pltpu.VMEM((K, H), jnp.float32),
        pltpu.VMEM((K,), jnp.int32),
        pltpu.VMEM((K,), jnp.int32),
        pltpu.VMEM((K,), jnp.int32),
        pltpu.VMEM((K,), jnp.int32),
        pltpu.VMEM((K,), jnp.int32),
        pltpu.VMEM((K,), jnp.int32),
        pltpu.SemaphoreType.DMA,
        pltpu.SemaphoreType.DMA,
        pltpu.SemaphoreType.DMA,
        pltpu.SemaphoreType.DMA,
        pltpu.SemaphoreType.DMA,
        pltpu.SemaphoreType.DMA,
    ],
)
def _sc_edge_pass(b_hbm, c_hbm, src_hbm, dst_hbm, z_hbm, out_hbm,
                  b_sh, agg_sh, cbuf0, cbuf1, gbuf0, gbuf1,
                  sidx0, sidx1, didx0, didx1, didx2, didx3,
                  semf0, semf1, semg0, semg1, sems0, sems1):
    c = lax.axis_index("c")
    s = lax.axis_index("s")
    row0 = s * NPT
    slots = ((cbuf0, gbuf0, sidx0, semf0, semg0, sems0),
             (cbuf1, gbuf1, sidx1, semf1, semg1, sems1))
    dring = (didx0, didx1, didx2, didx3)
    NQ = NB // 4

    def _fetch(e0, ru):
        slot = slots[ru % 2]
        return (pltpu.make_async_copy(src_hbm.at[pl.ds(e0, K)], slot[2],
                                      slot[3]),
                pltpu.make_async_copy(dst_hbm.at[pl.ds(e0, K)], dring[ru],
                                      slot[3]),
                pltpu.make_async_copy(
                    c_hbm.at[pl.ds(e0, K), pl.ds(c * H, H)], slot[0],
                    slot[3]))

    def _scatter_start(ru):
        slot = slots[ru % 2]
        pltpu.async_copy(slot[1], agg_sh.at[dring[ru]], slot[5], add=True)

    def _scatter_wait(ru):
        slot = slots[ru % 2]
        pltpu.make_async_copy(slot[1], agg_sh.at[dring[ru]], slot[5]).wait()

    pltpu.sync_copy(b_hbm.at[c, pl.ds(row0, NPT)], b_sh.at[pl.ds(row0, NPT)])
    pltpu.sync_copy(z_hbm, agg_sh.at[pl.ds(row0, NPT)])
    plsc.subcore_barrier()

    tile0 = s * EPT
    for cp in _fetch(tile0, 0) + _fetch(tile0 + K, 1):
        cp.start()

    def _relu_add(slot):
        cbuf, gbuf = slot[0], slot[1]

        def _mrow(r4, cr):
            r = r4 * 4
            for rr in range(4):
                for jj in range(H // 16):
                    sl = pl.ds(jj * 16, 16)
                    gbuf[r + rr, sl] = jnp.maximum(
                        gbuf[r + rr, sl] + cbuf[r + rr, sl], 0.0)
            return cr

        lax.fori_loop(0, K // 4, _mrow, 0)

    def _blk(q, carry):
        base = tile0 + q * 4 * K
        for u0 in (0, 2):
            gathers = []
            for u in (u0, u0 + 1):
                slot = slots[u % 2]
                for cp in _fetch(base + u * K, u):
                    cp.wait()
                if u0 == 0:
                    @pl.when(q > 0)
                    def _():
                        _scatter_wait(u + 2)
                else:
                    _scatter_wait(u - 2)
                gathers.append(
                    pltpu.async_copy(b_sh.at[slot[2]], slot[1], slot[4]))
            for u in (u0, u0 + 1):
                slot = slots[u % 2]
                gathers[u - u0].wait()
                _relu_add(slot)
                _scatter_start(u)
                if u0 == 0:
                    for cp in _fetch(base + (u + 2) * K, u + 2):
                        cp.start()
                else:
                    @pl.when(q < NQ - 1)
                    def _():
                        for cp in _fetch(base + (u + 2) * K, (u + 2) % 4):
                            cp.start()

        return carry

    lax.fori_loop(0, NQ, _blk, 0)
    _scatter_wait(2)
    _scatter_wait(3)

    plsc.subcore_barrier()
    pltpu.sync_copy(agg_sh.at[pl.ds(row0, NPT)],
                    out_hbm.at[c, pl.ds(row0, NPT)])


@functools.partial(
    pl.kernel,
    out_type=jax.ShapeDtypeStruct((2, N2, 16), jnp.float32),
    mesh=_mesh,
    compiler_params=_sc_params,
    scratch_types=[
        pltpu.VMEM_SHARED((N2, 16), jnp.float32),
        pltpu.VMEM((K, 16), jnp.float32),
        pltpu.VMEM((K,), jnp.int32),
        pltpu.VMEM((K,), jnp.int32),
        pltpu.SemaphoreType.DMA,
        pltpu.SemaphoreType.DMA,
    ],
)
def _sc_counts(dst2_hbm, z_hbm, ones_hbm, out_hbm, cnt_sh, ones,
               idx0, idx1, semi0, semi1):
    c = lax.axis_index("c")
    s = lax.axis_index("s")
    row0 = s * NPT
    slots = ((idx0, semi0), (idx1, semi1))
    NBH = NB // 2

    def _fetch(i, slot):
        e0 = s * EPT + i * K
        return pltpu.make_async_copy(dst2_hbm.at[c, pl.ds(e0, K)],
                                     slot[0], slot[1])

    pltpu.sync_copy(z_hbm, cnt_sh.at[pl.ds(row0, NPT)])
    pltpu.sync_copy(ones_hbm, ones)
    plsc.subcore_barrier()

    _fetch(0, slots[0]).start()
    _fetch(1, slots[1]).start()

    def _blk(j, carry):
        for t, slot in enumerate(slots):
            _fetch(2 * j + t, slot).wait()
            pltpu.sync_copy(ones, cnt_sh.at[slot[0]], add=True)

            @pl.when(j < NBH - 1)
            def _():
                _fetch(2 * j + 2 + t, slot).start()

        return carry

    lax.fori_loop(0, NBH, _blk, 0)

    plsc.subcore_barrier()
    pltpu.sync_copy(cnt_sh.at[pl.ds(row0, NPT)],
                    out_hbm.at[c, pl.ds(row0, NPT)])


def _dot(a, b):
    return jnp.dot(a, b, preferred_element_type=jnp.float32)


def _fold_body(ws_ref, wm_ref, wd_ref, wu_ref, wc_ref, wdu_ref):
    wc_ref[...] = _dot(ws_ref[:DM, :], wm_ref[:DM, :])
    wdu_ref[...] = _dot(wd_ref[:DM, :], wu_ref[DM:, :])


def _fold_weights(w_src, w_msg, w_dst, w_upd):
    return pl.pallas_call(
        _fold_body,
        out_shape=(jax.ShapeDtypeStruct((DM, DM), jnp.float32),
                   jax.ShapeDtypeStruct((DM, DM), jnp.float32)),
    )(w_src, w_msg, w_dst, w_upd)


_RN = 1024


def _node_setup_body(mem_ref, nf_ref, ws_ref, bs_ref, wm_ref, wd_ref,
                     bd_ref, wu_ref, bu_ref, wc_ref,
                     d_ref, f_ref, b_ref):
    nf = nf_ref[...]
    d = _dot(_dot(nf, ws_ref[DM:, :]) + bs_ref[...], wm_ref[:DM, :])
    d_ref[...] = d
    f_ref[...] = _dot(_dot(nf, wd_ref[DM:, :]) + bd_ref[...],
                      wu_ref[DM:, :]) + bu_ref[...]
    b0 = _dot(mem_ref[...], wc_ref[...]) + d
    b_ref[0] = b0[:, :H]
    b_ref[1] = b0[:, H:]


def _node_setup(mem_p, nf_p, w_src, b_src2, w_msg, w_dst, b_dst2, w_upd,
                b_upd2, wc):
    grid = (N2 // _RN,)
    row_spec = pl.BlockSpec((_RN, DM), lambda i: (i, 0))
    w_specs = [
        pl.BlockSpec((DM + DM, DM), lambda i: (0, 0)),
        pl.BlockSpec((1, DM), lambda i: (0, 0)),
        pl.BlockSpec((DM + DE + DT, DM), lambda i: (0, 0)),
        pl.BlockSpec((DM + DM, DM), lambda i: (0, 0)),
        pl.BlockSpec((1, DM), lambda i: (0, 0)),
        pl.BlockSpec((DM + DM, DM), lambda i: (0, 0)),
        pl.BlockSpec((1, DM), lambda i: (0, 0)),
        pl.BlockSpec((DM, DM), lambda i: (0, 0)),
    ]
    return pl.pallas_call(
        _node_setup_body,
        grid=grid,
        in_specs=[row_spec, row_spec] + w_specs,
        out_specs=(row_spec, row_spec,
                   pl.BlockSpec((2, _RN, H), lambda i: (0, i, 0))),
        out_shape=(jax.ShapeDtypeStruct((N2, DM), jnp.float32),
                   jax.ShapeDtypeStruct((N2, DM), jnp.float32),
                   jax.ShapeDtypeStruct((2, N2, H), jnp.float32)),
    )(mem_p, nf_p, w_src, b_src2, w_msg, w_dst, b_dst2, w_upd, b_upd2, wc)


_RE = 1600


def _edge_setup_body(ef_ref, te_ref, wm_ref, bm_ref, c_ref):
    c_ref[...] = (_dot(ef_ref[...], wm_ref[DM:DM + DE, :]) +
                  _dot(te_ref[...], wm_ref[DM + DE:, :]) + bm_ref[...])


def _edge_setup(ef, te, w_msg, b_msg2):
    grid = (E // _RE,)
    return pl.pallas_call(
        _edge_setup_body,
        grid=grid,
        in_specs=[
            pl.BlockSpec((_RE, DE), lambda i: (i, 0)),
            pl.BlockSpec((_RE, DT), lambda i: (i, 0)),
            pl.BlockSpec((DM + DE + DT, DM), lambda i: (0, 0)),
            pl.BlockSpec((1, DM), lambda i: (0, 0)),
        ],
        out_specs=pl.BlockSpec((_RE, DM), lambda i: (i, 0)),
        out_shape=jax.ShapeDtypeStruct((E2, DM), jnp.float32),
    )(ef, te, w_msg, b_msg2)


def _layer_body(agg_ref, cnt_ref, mem_ref, d_ref, f_ref, wu_ref, wdu_ref,
                wc_ref, mem_out_ref, b_out_ref):
    inv = 1.0 / jnp.maximum(cnt_ref[:, 0:1], 1.0)
    a0 = agg_ref[0] * inv
    a1 = agg_ref[1] * inv
    pre = (_dot(a0, wu_ref[:H, :]) + _dot(a1, wu_ref[H:DM, :]) +
           _dot(mem_ref[...], wdu_ref[...]) + f_ref[...])
    m2 = jnp.tanh(pre)
    mem_out_ref[...] = m2
    bn = _dot(m2, wc_ref[...]) + d_ref[...]
    b_out_ref[0] = bn[:, :H]
    b_out_ref[1] = bn[:, H:]


def _layer_update(agg, cnt_l, mem_p, d, f, w_upd, wdu, wc):
    grid = (N2 // _RN,)
    row_spec = pl.BlockSpec((_RN, DM), lambda i: (i, 0))
    return pl.pallas_call(
        _layer_body,
        grid=grid,
        in_specs=[
            pl.BlockSpec((2, _RN, H), lambda i: (0, i, 0)),
            pl.BlockSpec((_RN, 16), lambda i: (i, 0)),
            row_spec,
            row_spec,
            row_spec,
            pl.BlockSpec((DM + DM, DM), lambda i: (0, 0)),
            pl.BlockSpec((DM, DM), lambda i: (0, 0)),
            pl.BlockSpec((DM, DM), lambda i: (0, 0)),
        ],
        out_specs=(row_spec, pl.BlockSpec((2, _RN, H), lambda i: (0, i, 0))),
        out_shape=(jax.ShapeDtypeStruct((N2, DM), jnp.float32),
                   jax.ShapeDtypeStruct((2, N2, H), jnp.float32)),
    )(agg, cnt_l, mem_p, d, f, w_upd, wdu, wc)


def kernel(node_memory, node_features, edge_features, time_encoding,
           edge_index_causal, edge_index_conseq,
           W_src, b_src, W_msg, b_msg, W_dst, b_dst, W_upd, b_upd):
    pad = ((0, N2 - N), (0, 0))
    mem_p = jnp.pad(node_memory, pad)
    nf_p = jnp.pad(node_features, pad)
    b_src2 = b_src.reshape(1, DM)
    b_msg2 = b_msg.reshape(1, DM)
    b_dst2 = b_dst.reshape(1, DM)
    b_upd2 = b_upd.reshape(1, DM)

    epad = E2 - E
    src_c = jnp.concatenate([edge_index_causal[0],
                             jnp.zeros((epad,), jnp.int32)])
    dst_c = jnp.concatenate([edge_index_causal[1],
                             jnp.full((epad,), N, jnp.int32)])
    src_q = jnp.concatenate([edge_index_conseq[0],
                             jnp.zeros((epad,), jnp.int32)])
    dst_q = jnp.concatenate([edge_index_conseq[1],
                             jnp.full((epad,), N, jnp.int32)])
    zeros_nh = jnp.zeros((NPT, H), jnp.float32)
    zeros_n16 = jnp.zeros((NPT, 16), jnp.float32)
    ones_k16 = jnp.ones((K, 16), jnp.float32)

    wc, wdu = _fold_weights(W_src, W_msg, W_dst, W_upd)
    d, f, b = _node_setup(mem_p, nf_p, W_src, b_src2, W_msg, W_dst, b_dst2,
                          W_upd, b_upd2, wc)
    c_edge = _edge_setup(edge_features, time_encoding, W_msg, b_msg2)
    dst2 = jnp.stack([dst_c, dst_q])
    cnt = _sc_counts(dst2, zeros_n16, ones_k16)

    for layer in range(4):
        if layer % 2 == 0:
            src_l, dst_l = src_c, dst_c
        else:
            src_l, dst_l = src_q, dst_q
        agg = _sc_edge_pass(b, c_edge, src_l, dst_l, zeros_nh)
        mem_p, b = _layer_update(agg, cnt[layer % 2], mem_p, d, f, W_upd,
                                 wdu, wc)
    return mem_p[:N]
